# Initial kernel scaffold; baseline (speedup 1.0000x reference)
import functools

import jax
import jax.numpy as jnp
from jax import lax
from jax.experimental import pallas as pl
from jax.experimental.pallas import tpu as pltpu
from jax.experimental.pallas import tpu_sc as plsc

N = 10000
E = 320000
F = 128
C = 40
CP = 48
R = 10240
NC = 2
NS = 16
NW = NC * NS
B = 128
KCH = 79
EPW = KCH * B
ROWS_PT = R // NS
RB = 1024
GRID = R // RB

_mesh = plsc.VectorSubcoreMesh(core_axis_name="c", subcore_axis_name="s")


@functools.partial(
    pl.kernel,
    out_type=jax.ShapeDtypeStruct((NC, R, 8), jnp.float32),
    mesh=_mesh,
    scratch_types=[
        pltpu.VMEM_SHARED((R, 8), jnp.float32),
        pltpu.VMEM((ROWS_PT, 8), jnp.float32),
        pltpu.VMEM((KCH, B), jnp.int32),
        pltpu.VMEM((B, 8), jnp.float32),
    ],
)
def _deg_kernel(dst_hbm, zeros_hbm, ones_hbm, out_hbm, acc, zbuf, didx, ones_v):
    c = lax.axis_index("c")
    s = lax.axis_index("s")
    wid = c * NS + s
    pltpu.sync_copy(zeros_hbm, zbuf)
    pltpu.sync_copy(zbuf, acc.at[pl.ds(s * ROWS_PT, ROWS_PT)])
    pltpu.sync_copy(ones_hbm, ones_v)
    pltpu.sync_copy(dst_hbm.at[wid], didx)
    plsc.subcore_barrier()

    def body(j, carry):
        pltpu.sync_copy(ones_v, acc.at[didx.at[j]], add=True)
        return carry

    lax.fori_loop(0, KCH, body, 0)
    plsc.subcore_barrier()
    pltpu.sync_copy(acc.at[pl.ds(s * ROWS_PT, ROWS_PT)],
                    out_hbm.at[c, pl.ds(s * ROWS_PT, ROWS_PT)])


@functools.partial(
    pl.kernel,
    out_type=jax.ShapeDtypeStruct((NC, R, CP), jnp.float32),
    mesh=_mesh,
    scratch_types=[
        pltpu.VMEM_SHARED((R, CP), jnp.float32),
        pltpu.VMEM((ROWS_PT, CP), jnp.float32),
        pltpu.VMEM((KCH, B), jnp.int32),
        pltpu.VMEM((KCH, B), jnp.int32),
        pltpu.VMEM((B, CP), jnp.float32),
        pltpu.SemaphoreType.DMA,
    ],
)
def _prop_kernel(y_hbm, src_hbm, dst_hbm, zeros_hbm, out_hbm,
                 acc, zbuf, sidx, didx, rows, sem):
    c = lax.axis_index("c")
    s = lax.axis_index("s")
    wid = c * NS + s
    pltpu.sync_copy(zeros_hbm, zbuf)
    pltpu.sync_copy(zbuf, acc.at[pl.ds(s * ROWS_PT, ROWS_PT)])
    pltpu.sync_copy(src_hbm.at[wid], sidx)
    pltpu.sync_copy(dst_hbm.at[wid], didx)
    plsc.subcore_barrier()

    def body(j, carry):
        pltpu.async_copy(y_hbm.at[sidx.at[j]], rows, sem).wait()
        pltpu.sync_copy(rows, acc.at[didx.at[j]], add=True)
        return carry

    lax.fori_loop(0, KCH, body, 0)
    plsc.subcore_barrier()
    pltpu.sync_copy(acc.at[pl.ds(s * ROWS_PT, ROWS_PT)],
                    out_hbm.at[c, pl.ds(s * ROWS_PT, ROWS_PT)])


def _tc1_body(x_ref, w_ref, deg_ref, y1_ref, norm_ref):
    d = deg_ref[0, :, :1] + deg_ref[1, :, :1]
    norm = jnp.where(d > 0.0, lax.rsqrt(jnp.maximum(d, 1.0)), 0.0)
    norm_ref[...] = norm
    y1_ref[...] = jnp.dot(x_ref[...], w_ref[...],
                          preferred_element_type=jnp.float32) * norm


_tc1 = pl.pallas_call(
    _tc1_body,
    grid=(GRID,),
    in_specs=[
        pl.BlockSpec((RB, F), lambda i: (i, 0)),
        pl.BlockSpec((F, CP), lambda i: (0, 0)),
        pl.BlockSpec((2, RB, 8), lambda i: (0, i, 0)),
    ],
    out_specs=[
        pl.BlockSpec((RB, CP), lambda i: (i, 0)),
        pl.BlockSpec((RB, 1), lambda i: (i, 0)),
    ],
    out_shape=[
        jax.ShapeDtypeStruct((R, CP), jnp.float32),
        jax.ShapeDtypeStruct((R, 1), jnp.float32),
    ],
)


def _tc2_body(z_ref, norm_ref, y2_ref):
    n = norm_ref[...]
    y2_ref[...] = (z_ref[0] + z_ref[1]) * (n * n)


_tc2 = pl.pallas_call(
    _tc2_body,
    grid=(GRID,),
    in_specs=[
        pl.BlockSpec((2, RB, CP), lambda i: (0, i, 0)),
        pl.BlockSpec((RB, 1), lambda i: (i, 0)),
    ],
    out_specs=pl.BlockSpec((RB, CP), lambda i: (i, 0)),
    out_shape=jax.ShapeDtypeStruct((R, CP), jnp.float32),
)


def _tc3_body(z_ref, norm_ref, b_ref, out_ref):
    out_ref[...] = (z_ref[0] + z_ref[1]) * norm_ref[...] + b_ref[...]


_tc3 = pl.pallas_call(
    _tc3_body,
    grid=(GRID,),
    in_specs=[
        pl.BlockSpec((2, RB, CP), lambda i: (0, i, 0)),
        pl.BlockSpec((RB, 1), lambda i: (i, 0)),
        pl.BlockSpec((1, CP), lambda i: (0, 0)),
    ],
    out_specs=pl.BlockSpec((RB, CP), lambda i: (i, 0)),
    out_shape=jax.ShapeDtypeStruct((R, CP), jnp.float32),
)


def kernel(features, edge_index, W, b):
    src = edge_index[0].astype(jnp.int32)
    dst = edge_index[1].astype(jnp.int32)
    pad = NW * EPW - E
    src_p = jnp.concatenate([src, jnp.zeros((pad,), jnp.int32)]).reshape(NW, KCH, B)
    dst_p = jnp.concatenate([dst, jnp.full((pad,), N, jnp.int32)]).reshape(NW, KCH, B)
    xp = jnp.pad(features, ((0, R - N), (0, 0)))
    Wp = jnp.pad(W, ((0, 0), (0, CP - C)))
    bp = jnp.pad(b, (0, CP - C)).reshape(1, CP)
    zeros48 = jnp.zeros((ROWS_PT, CP), jnp.float32)
    zeros8 = jnp.zeros((ROWS_PT, 8), jnp.float32)
    ones8 = jnp.ones((B, 8), jnp.float32)

    degp = _deg_kernel(dst_p, zeros8, ones8)
    y1, norm = _tc1(xp, Wp, degp)
    z1 = _prop_kernel(y1, src_p, dst_p, zeros48)
    y2 = _tc2(z1, norm)
    z2 = _prop_kernel(y2, src_p, dst_p, zeros48)
    outp = _tc3(z2, norm, bp)
    return outp[:N, :C]


# speedup vs baseline: 8.6563x; 8.6563x over previous
import functools

import jax
import jax.numpy as jnp
from jax import lax
from jax.experimental import pallas as pl
from jax.experimental.pallas import tpu as pltpu
from jax.experimental.pallas import tpu_sc as plsc

N = 10000
E = 320000
F = 128
C = 40
CP = 48
R = 10240
NC = 2
NS = 16
NW = NC * NS
B = 128
KCH = 79
EPW = KCH * B
ROWS_PT = R // NS
RB = 1024
GRID = R // RB

_mesh = plsc.VectorSubcoreMesh(core_axis_name="c", subcore_axis_name="s")
_sc_params = pltpu.CompilerParams(use_tc_tiling_on_sc=False)


@functools.partial(
    pl.kernel,
    out_type=jax.ShapeDtypeStruct((NC, R, 8), jnp.float32),
    mesh=_mesh,
    scratch_types=[
        pltpu.VMEM_SHARED((R, 8), jnp.float32),
        pltpu.VMEM((ROWS_PT, 8), jnp.float32),
        pltpu.VMEM((KCH, B), jnp.int32),
        pltpu.VMEM((B, 8), jnp.float32),
    ],
    compiler_params=_sc_params,
)
def _deg_kernel(dst_hbm, zeros_hbm, ones_hbm, out_hbm, acc, zbuf, didx, ones_v):
    c = lax.axis_index("c")
    s = lax.axis_index("s")
    wid = c * NS + s
    pltpu.sync_copy(zeros_hbm, zbuf)
    pltpu.sync_copy(zbuf, acc.at[pl.ds(s * ROWS_PT, ROWS_PT)])
    pltpu.sync_copy(ones_hbm, ones_v)
    pltpu.sync_copy(dst_hbm.at[wid], didx)
    plsc.subcore_barrier()

    def body(j, carry):
        pltpu.sync_copy(ones_v, acc.at[didx.at[j]], add=True)
        return carry

    lax.fori_loop(0, KCH, body, 0)
    plsc.subcore_barrier()
    pltpu.sync_copy(acc.at[pl.ds(s * ROWS_PT, ROWS_PT)],
                    out_hbm.at[c, pl.ds(s * ROWS_PT, ROWS_PT)])


@functools.partial(
    pl.kernel,
    out_type=jax.ShapeDtypeStruct((NC, R, CP), jnp.float32),
    mesh=_mesh,
    scratch_types=[
        pltpu.VMEM_SHARED((R, CP), jnp.float32),
        pltpu.VMEM((ROWS_PT, CP), jnp.float32),
        pltpu.VMEM((KCH, B), jnp.int32),
        pltpu.VMEM((KCH, B), jnp.int32),
        pltpu.VMEM((B, CP), jnp.float32),
        pltpu.SemaphoreType.DMA,
    ],
    compiler_params=_sc_params,
)
def _prop_kernel(y_hbm, src_hbm, dst_hbm, zeros_hbm, out_hbm,
                 acc, zbuf, sidx, didx, rows, sem):
    c = lax.axis_index("c")
    s = lax.axis_index("s")
    wid = c * NS + s
    pltpu.sync_copy(zeros_hbm, zbuf)
    pltpu.sync_copy(zbuf, acc.at[pl.ds(s * ROWS_PT, ROWS_PT)])
    pltpu.sync_copy(src_hbm.at[wid], sidx)
    pltpu.sync_copy(dst_hbm.at[wid], didx)
    plsc.subcore_barrier()

    def body(j, carry):
        pltpu.async_copy(y_hbm.at[sidx.at[j]], rows, sem).wait()
        pltpu.sync_copy(rows, acc.at[didx.at[j]], add=True)
        return carry

    lax.fori_loop(0, KCH, body, 0)
    plsc.subcore_barrier()
    pltpu.sync_copy(acc.at[pl.ds(s * ROWS_PT, ROWS_PT)],
                    out_hbm.at[c, pl.ds(s * ROWS_PT, ROWS_PT)])


def _tc1_body(x_ref, w_ref, deg_ref, y1_ref, norm_ref):
    d = deg_ref[0, :, :1] + deg_ref[1, :, :1]
    norm = jnp.where(d > 0.0, lax.rsqrt(jnp.maximum(d, 1.0)), 0.0)
    norm_ref[...] = norm
    y1_ref[...] = jnp.dot(x_ref[...], w_ref[...],
                          preferred_element_type=jnp.float32) * norm


_tc1 = pl.pallas_call(
    _tc1_body,
    grid=(GRID,),
    in_specs=[
        pl.BlockSpec((RB, F), lambda i: (i, 0)),
        pl.BlockSpec((F, CP), lambda i: (0, 0)),
        pl.BlockSpec((2, RB, 8), lambda i: (0, i, 0)),
    ],
    out_specs=[
        pl.BlockSpec((RB, CP), lambda i: (i, 0)),
        pl.BlockSpec((RB, 1), lambda i: (i, 0)),
    ],
    out_shape=[
        jax.ShapeDtypeStruct((R, CP), jnp.float32),
        jax.ShapeDtypeStruct((R, 1), jnp.float32),
    ],
)


def _tc2_body(z_ref, norm_ref, y2_ref):
    n = norm_ref[...]
    y2_ref[...] = (z_ref[0] + z_ref[1]) * (n * n)


_tc2 = pl.pallas_call(
    _tc2_body,
    grid=(GRID,),
    in_specs=[
        pl.BlockSpec((2, RB, CP), lambda i: (0, i, 0)),
        pl.BlockSpec((RB, 1), lambda i: (i, 0)),
    ],
    out_specs=pl.BlockSpec((RB, CP), lambda i: (i, 0)),
    out_shape=jax.ShapeDtypeStruct((R, CP), jnp.float32),
)


def _tc3_body(z_ref, norm_ref, b_ref, out_ref):
    out_ref[...] = (z_ref[0] + z_ref[1]) * norm_ref[...] + b_ref[...]


_tc3 = pl.pallas_call(
    _tc3_body,
    grid=(GRID,),
    in_specs=[
        pl.BlockSpec((2, RB, CP), lambda i: (0, i, 0)),
        pl.BlockSpec((RB, 1), lambda i: (i, 0)),
        pl.BlockSpec((1, CP), lambda i: (0, 0)),
    ],
    out_specs=pl.BlockSpec((RB, CP), lambda i: (i, 0)),
    out_shape=jax.ShapeDtypeStruct((R, CP), jnp.float32),
)


def kernel(features, edge_index, W, b):
    src = edge_index[0].astype(jnp.int32)
    dst = edge_index[1].astype(jnp.int32)
    pad = NW * EPW - E
    src_p = jnp.concatenate([src, jnp.zeros((pad,), jnp.int32)]).reshape(NW, KCH, B)
    dst_p = jnp.concatenate([dst, jnp.full((pad,), N, jnp.int32)]).reshape(NW, KCH, B)
    xp = jnp.pad(features, ((0, R - N), (0, 0)))
    Wp = jnp.pad(W, ((0, 0), (0, CP - C)))
    bp = jnp.pad(b, (0, CP - C)).reshape(1, CP)
    zeros48 = jnp.zeros((ROWS_PT, CP), jnp.float32)
    zeros8 = jnp.zeros((ROWS_PT, 8), jnp.float32)
    ones8 = jnp.ones((B, 8), jnp.float32)

    degp = _deg_kernel(dst_p, zeros8, ones8)
    y1, norm = _tc1(xp, Wp, degp)
    z1 = _prop_kernel(y1, src_p, dst_p, zeros48)
    y2 = _tc2(z1, norm)
    z2 = _prop_kernel(y2, src_p, dst_p, zeros48)
    outp = _tc3(z2, norm, bp)
    return outp[:N, :C]
